# baseline (speedup 1.0000x reference)
import jax
import jax.numpy as jnp
from jax.experimental import pallas as pl

CHUNKS = 8
RES_DIM = 2048
BIAS = 1.6
C1, C3, C5 = 1.0, -1.0 / 3.0, 2.0 / 15.0

TI = 512


def _body(pv_ref, s_ref, wr_ref, out_ref):
    w = wr_ref[0]
    s = s_ref[0]
    pre = jnp.dot(w, s, preferred_element_type=jnp.float32)[:, 0]
    pre = pre + pv_ref[0, 0, 0] + BIAS
    p2 = pre * pre
    out_ref[0, 0, 0] = pre * (C1 + p2 * (C3 + p2 * C5))


def kernel(proj_vars, res_state, wr):
    nb = RES_DIM // TI
    s2 = res_state[:, :, None]
    pv = proj_vars.reshape(CHUNKS, nb, 1, TI)
    out = pl.pallas_call(
        _body,
        grid=(CHUNKS, nb),
        in_specs=[
            pl.BlockSpec((1, 1, 1, TI), lambda c, i: (c, i, 0, 0)),
            pl.BlockSpec((1, RES_DIM, 1), lambda c, i: (c, 0, 0)),
            pl.BlockSpec((1, TI, RES_DIM), lambda c, i: (c, i, 0)),
        ],
        out_specs=pl.BlockSpec((1, 1, 1, TI), lambda c, i: (c, i, 0, 0)),
        out_shape=jax.ShapeDtypeStruct((CHUNKS, nb, 1, TI), jnp.float32),
    )(pv, s2, wr)
    return out.reshape(CHUNKS, RES_DIM)


# speedup vs baseline: 1.1878x; 1.1878x over previous
import jax
import jax.numpy as jnp
from jax.experimental import pallas as pl

CHUNKS = 8
RES_DIM = 2048
BIAS = 1.6
C1, C3, C5 = 1.0, -1.0 / 3.0, 2.0 / 15.0

TI = 512


def _body(pv_ref, s_ref, wr_ref, out_ref):
    w = wr_ref[0]
    s = s_ref[0]
    pre = jnp.sum(w * s, axis=1)
    pre = pre + pv_ref[0, 0, 0] + BIAS
    p2 = pre * pre
    out_ref[0, 0, 0] = pre * (C1 + p2 * (C3 + p2 * C5))


def kernel(proj_vars, res_state, wr):
    nb = RES_DIM // TI
    s2 = res_state[:, None, :]
    pv = proj_vars.reshape(CHUNKS, nb, 1, TI)
    out = pl.pallas_call(
        _body,
        grid=(CHUNKS, nb),
        in_specs=[
            pl.BlockSpec((1, 1, 1, TI), lambda c, i: (c, i, 0, 0)),
            pl.BlockSpec((1, 1, RES_DIM), lambda c, i: (c, 0, 0)),
            pl.BlockSpec((1, TI, RES_DIM), lambda c, i: (c, i, 0)),
        ],
        out_specs=pl.BlockSpec((1, 1, 1, TI), lambda c, i: (c, i, 0, 0)),
        out_shape=jax.ShapeDtypeStruct((CHUNKS, nb, 1, TI), jnp.float32),
    )(pv, s2, wr)
    return out.reshape(CHUNKS, RES_DIM)


# speedup vs baseline: 1.4179x; 1.1937x over previous
import jax
import jax.numpy as jnp
from jax.experimental import pallas as pl

CHUNKS = 8
RES_DIM = 2048
BIAS = 1.6
C1, C3, C5 = 1.0, -1.0 / 3.0, 2.0 / 15.0

TI = 1024


def _body(pv_ref, s_ref, wr_ref, out_ref):
    w = wr_ref[0]
    s = s_ref[0]
    pre = jnp.sum(w * s, axis=1)
    pre = pre + pv_ref[0, 0, 0] + BIAS
    p2 = pre * pre
    out_ref[0, 0, 0] = pre * (C1 + p2 * (C3 + p2 * C5))


def kernel(proj_vars, res_state, wr):
    nb = RES_DIM // TI
    s2 = res_state[:, None, :]
    pv = proj_vars.reshape(CHUNKS, nb, 1, TI)
    out = pl.pallas_call(
        _body,
        grid=(CHUNKS, nb),
        in_specs=[
            pl.BlockSpec((1, 1, 1, TI), lambda c, i: (c, i, 0, 0)),
            pl.BlockSpec((1, 1, RES_DIM), lambda c, i: (c, 0, 0)),
            pl.BlockSpec((1, TI, RES_DIM), lambda c, i: (c, i, 0)),
        ],
        out_specs=pl.BlockSpec((1, 1, 1, TI), lambda c, i: (c, i, 0, 0)),
        out_shape=jax.ShapeDtypeStruct((CHUNKS, nb, 1, TI), jnp.float32),
    )(pv, s2, wr)
    return out.reshape(CHUNKS, RES_DIM)
